# baseline (speedup 1.0000x reference)
import jax
import jax.numpy as jnp
from jax import lax
from jax.experimental import pallas as pl
from jax.experimental.pallas import tpu as pltpu
from jax.experimental.pallas import tpu_sc as plsc

BATCH = 4
SEQ = 4096
EMB = 1024
NUM_TOK = BATCH * SEQ

_info = plsc.get_sparse_core_info()
NC, NS = _info.num_cores, _info.num_subcores
NW = NC * NS
W_PER_B = NW // BATCH

PER_W = NUM_TOK // NW
CHUNK = 32
NCHUNK = PER_W // CHUNK
NBUF = 3
DEPTH = 2


def _body(pe_hbm, tok_hbm, out_hbm, idx_v, buf, gsem, ssem):
    wid = lax.axis_index("s") * NC + lax.axis_index("c")
    b = wid // W_PER_B
    l_base = (wid % W_PER_B) * PER_W

    PRE = 128
    pltpu.sync_copy(tok_hbm.at[b, pl.ds(l_base, PRE)], idx_v.at[pl.ds(0, PRE)])

    def _gather_desc(t, r):
        off = pl.multiple_of(t * CHUNK, 8)
        return pltpu.make_async_copy(
            pe_hbm.at[idx_v.at[pl.ds(off, CHUNK)]], buf.at[r], gsem.at[r]
        )

    def _store_desc(t, r):
        return pltpu.make_async_copy(
            buf.at[r], out_hbm.at[b, pl.ds(l_base + t * CHUNK, CHUNK)], ssem.at[r]
        )

    def start_gather(t, r):
        _gather_desc(t, r).start()

    def start_store(t, r):
        _store_desc(t, r).start()

    for t in range(DEPTH):
        start_gather(t, t % NBUF)
    pltpu.sync_copy(
        tok_hbm.at[b, pl.ds(l_base + PRE, PER_W - PRE)],
        idx_v.at[pl.ds(PRE, PER_W - PRE)],
    )

    @pl.loop(0, NCHUNK)
    def _step(t):
        tf = t + DEPTH
        rf = lax.rem(tf, NBUF)

        @pl.when(tf < NCHUNK)
        def _():
            @pl.when(tf >= NBUF)
            def _():
                _store_desc(0, rf).wait()

            start_gather(tf, rf)

        r = lax.rem(t, NBUF)
        _gather_desc(0, r).wait()
        start_store(t, r)

    for t in range(NCHUNK - NBUF, NCHUNK):
        _store_desc(0, t % NBUF).wait()


@jax.jit
def kernel(tok, pe):
    return pl.kernel(
        _body,
        out_type=jax.ShapeDtypeStruct((BATCH, SEQ, EMB), jnp.float32),
        mesh=plsc.VectorSubcoreMesh(core_axis_name="c", subcore_axis_name="s"),
        scratch_types=[
            pltpu.VMEM((PER_W,), jnp.int32),
            pltpu.VMEM((NBUF, CHUNK, EMB), jnp.float32),
            pltpu.SemaphoreType.DMA((NBUF,)),
            pltpu.SemaphoreType.DMA((NBUF,)),
        ],
    )(pe, tok.astype(jnp.int32))
